# Initial kernel scaffold; baseline (speedup 1.0000x reference)
import jax
import jax.numpy as jnp
from jax.experimental import pallas as pl


def kernel(adj, n_feat, lin0_w, lin0_b, gin_w, gin_b, gru_w_ih, gru_w_hh, gru_b_ih, gru_b_hh):
    raise NotImplementedError("write your pallas kernel here")


# speedup vs baseline: 1.3267x; 1.3267x over previous
import functools

import jax
import jax.numpy as jnp
from jax.experimental import pallas as pl

STEPS = 2


def _encoder_kernel(adj_ref, x_ref, lin0_w_ref, lin0_b_ref, gin_w_ref,
                    gin_b_ref, w_ih_ref, w_hh_ref, b_ih_ref, b_hh_ref,
                    out_ref):
    f32 = jnp.float32
    adj = adj_ref[0]
    x = x_ref[0]

    out = jax.nn.relu(
        jax.lax.dot_general(x, lin0_w_ref[...],
                            (((1,), (1,)), ((), ())),
                            preferred_element_type=f32)
        + lin0_b_ref[...])
    h = out

    adj_b = adj.astype(jnp.bfloat16)
    H = out.shape[-1]

    for _ in range(STEPS):
        agg = jax.lax.dot_general(adj_b, out.astype(jnp.bfloat16),
                                  (((0,), (0,)), ((), ())),
                                  preferred_element_type=f32)
        m = jax.nn.relu(
            jax.lax.dot_general(out + agg, gin_w_ref[...],
                                (((1,), (1,)), ((), ())),
                                preferred_element_type=f32)
            + gin_b_ref[...])
        gi = jax.lax.dot_general(m, w_ih_ref[...],
                                 (((1,), (1,)), ((), ())),
                                 preferred_element_type=f32) + b_ih_ref[...]
        gh = jax.lax.dot_general(h, w_hh_ref[...],
                                 (((1,), (1,)), ((), ())),
                                 preferred_element_type=f32) + b_hh_ref[...]
        r = jax.nn.sigmoid(gi[:, :H] + gh[:, :H])
        z = jax.nn.sigmoid(gi[:, H:2 * H] + gh[:, H:2 * H])
        n = jnp.tanh(gi[:, 2 * H:] + r * gh[:, 2 * H:])
        out = (1.0 - z) * n + z * h
        h = out

    out_ref[0] = out


def kernel(adj, n_feat, lin0_w, lin0_b, gin_w, gin_b, gru_w_ih, gru_w_hh,
           gru_b_ih, gru_b_hh):
    B, N, FT = n_feat.shape
    H = lin0_w.shape[0]

    full = lambda shape: pl.BlockSpec(shape, lambda b: (0,) * len(shape))
    out3 = pl.pallas_call(
        _encoder_kernel,
        grid=(B,),
        in_specs=[
            pl.BlockSpec((1, N, N), lambda b: (b, 0, 0)),
            pl.BlockSpec((1, N, FT), lambda b: (b, 0, 0)),
            full((H, FT)),
            full((1, H)),
            full((H, H)),
            full((1, H)),
            full((3 * H, H)),
            full((3 * H, H)),
            full((1, 3 * H)),
            full((1, 3 * H)),
        ],
        out_specs=pl.BlockSpec((1, N, H), lambda b: (b, 0, 0)),
        out_shape=jax.ShapeDtypeStruct((B, N, H), jnp.float32),
    )(adj, n_feat, lin0_w, lin0_b.reshape(1, H), gin_w, gin_b.reshape(1, H),
      gru_w_ih, gru_w_hh, gru_b_ih.reshape(1, 3 * H),
      gru_b_hh.reshape(1, 3 * H))
    return out3.reshape(B * N, H)


# speedup vs baseline: 1.8604x; 1.4023x over previous
import jax
import jax.numpy as jnp
from jax.experimental import pallas as pl

STEPS = 2


def _encoder_kernel(adj_ref, x_ref, lin0_w_ref, lin0_b_ref, gin_w_ref,
                    gin_b_ref, w_ih_ref, w_hh_ref, b_ih_ref, b_hh_ref,
                    out_ref):
    f32 = jnp.float32
    bf16 = jnp.bfloat16
    adj_b = adj_ref[0].astype(bf16)
    x = x_ref[0]

    out_t = jax.nn.relu(
        jax.lax.dot_general(lin0_w_ref[...], x,
                            (((1,), (1,)), ((), ())),
                            preferred_element_type=f32)
        + lin0_b_ref[...])
    h_t = out_t
    H = out_t.shape[0]

    for _ in range(STEPS):
        agg_t = jax.lax.dot_general(out_t.astype(bf16), adj_b,
                                    (((1,), (0,)), ((), ())),
                                    preferred_element_type=f32)
        m_t = jax.nn.relu(
            jax.lax.dot_general(gin_w_ref[...], out_t + agg_t,
                                (((1,), (0,)), ((), ())),
                                preferred_element_type=f32)
            + gin_b_ref[...])
        gi = jax.lax.dot_general(w_ih_ref[...], m_t,
                                 (((1,), (0,)), ((), ())),
                                 preferred_element_type=f32) + b_ih_ref[...]
        gh = jax.lax.dot_general(w_hh_ref[...], h_t,
                                 (((1,), (0,)), ((), ())),
                                 preferred_element_type=f32) + b_hh_ref[...]
        r = jax.nn.sigmoid(gi[:H] + gh[:H])
        z = jax.nn.sigmoid(gi[H:2 * H] + gh[H:2 * H])
        n = jnp.tanh(gi[2 * H:] + r * gh[2 * H:])
        out_t = (1.0 - z) * n + z * h_t
        h_t = out_t

    out_ref[0] = out_t


def kernel(adj, n_feat, lin0_w, lin0_b, gin_w, gin_b, gru_w_ih, gru_w_hh,
           gru_b_ih, gru_b_hh):
    B, N, FT = n_feat.shape
    H = lin0_w.shape[0]

    full = lambda shape: pl.BlockSpec(shape, lambda b: (0,) * len(shape))
    out3 = pl.pallas_call(
        _encoder_kernel,
        grid=(B,),
        in_specs=[
            pl.BlockSpec((1, N, N), lambda b: (b, 0, 0)),
            pl.BlockSpec((1, N, FT), lambda b: (b, 0, 0)),
            full((H, FT)),
            full((H, 1)),
            full((H, H)),
            full((H, 1)),
            full((3 * H, H)),
            full((3 * H, H)),
            full((3 * H, 1)),
            full((3 * H, 1)),
        ],
        out_specs=pl.BlockSpec((1, H, N), lambda b: (b, 0, 0)),
        out_shape=jax.ShapeDtypeStruct((B, H, N), jnp.float32),
    )(adj, n_feat, lin0_w, lin0_b.reshape(H, 1), gin_w, gin_b.reshape(H, 1),
      gru_w_ih, gru_w_hh, gru_b_ih.reshape(3 * H, 1),
      gru_b_hh.reshape(3 * H, 1))
    return out3.transpose(0, 2, 1).reshape(B * N, H)
